# baseline (speedup 1.0000x reference)
import functools

import jax
import jax.numpy as jnp
from jax import lax
from jax.experimental import pallas as pl
from jax.experimental.pallas import tpu as pltpu
from jax.experimental.pallas import tpu_sc as plsc

VOCAB = 1000000
EMBED = 64
BATCH = 4096

_NC = 2
_NS = 16
_NW = _NC * _NS
_BPW = BATCH // _NW
_L = 16
_CH = 4
_NCH = 8


def _gather_one_table(wt_hbm, idx_hbm, out_hbm, base, idx_v, off_v, lane_v,
                      bufs, rows_v, sem):
    pltpu.sync_copy(idx_hbm.at[pl.ds(base, _BPW)], idx_v)
    lanes16 = lax.iota(jnp.int32, _L)

    for i in range(_BPW // _L):
        v = idx_v[pl.ds(i * _L, _L)]
        off_v[pl.ds(i * _L, _L)] = (v >> 7) << 7
        lane_v[pl.ds(i * _L, _L)] = v & 127

    def fire(j, c):
        vo = off_v[pl.ds(j * 32 + (c // 4) * _L, _L)]
        for k in range(_CH):
            off = pl.multiple_of(vo[(c % 4) * _CH + k], 128)
            pltpu.async_copy(wt_hbm.at[:, pl.ds(off, 128)],
                             bufs.at[(c % 3) * _CH + k], sem)

    def run(j, _):
        fire(j, 0)
        fire(j, 1)
        for c in range(_NCH):
            if c + 2 < _NCH:
                fire(j, c + 2)
            for k in range(_CH):
                pltpu.make_async_copy(wt_hbm.at[:, pl.ds(0, 128)],
                                      bufs.at[(c % 3) * _CH + k], sem).wait()
            vl = lane_v[pl.ds(j * 32 + (c // 4) * _L, _L)]
            for k in range(_CH):
                i = j * 32 + c * _CH + k
                lane = jnp.full((_L,), vl[(c % 4) * _CH + k], jnp.int32)
                slot = jnp.full((_L,), (c % 3) * _CH + k, jnp.int32)
                for q in range(EMBED // _L):
                    vals = plsc.load_gather(
                        bufs, [slot, lanes16 + q * _L, lane])
                    rows_v[i, pl.ds(q * _L, _L)] = vals
        return ()

    lax.fori_loop(0, _BPW // 32, run, ())
    pltpu.sync_copy(rows_v, out_hbm.at[pl.ds(base, _BPW)])


@functools.partial(
    pl.kernel,
    out_type=(
        jax.ShapeDtypeStruct((BATCH, EMBED), jnp.float32),
        jax.ShapeDtypeStruct((BATCH, EMBED), jnp.float32),
    ),
    mesh=plsc.VectorSubcoreMesh(core_axis_name="c", subcore_axis_name="s"),
    compiler_params=pltpu.CompilerParams(needs_layout_passes=False),
    scratch_types=[
        pltpu.VMEM((_BPW,), jnp.int32),
        pltpu.VMEM((_BPW,), jnp.int32),
        pltpu.VMEM((_BPW,), jnp.int32),
        pltpu.VMEM((3 * _CH, EMBED, 128), jnp.float32),
        pltpu.VMEM((_BPW, EMBED), jnp.float32),
        pltpu.SemaphoreType.DMA,
    ],
)
def _sc_gather(wct_hbm, ci_hbm, wxt_hbm, xi_hbm, out_c, out_x,
               idx_v, off_v, lane_v, bufs, rows_v, sem):
    wid = lax.axis_index("s") * _NC + lax.axis_index("c")
    base = wid * _BPW
    _gather_one_table(wct_hbm, ci_hbm, out_c, base, idx_v, off_v, lane_v,
                      bufs, rows_v, sem)
    _gather_one_table(wxt_hbm, xi_hbm, out_x, base, idx_v, off_v, lane_v,
                      bufs, rows_v, sem)


_TM = 512


def _mm_body(a_ref, b_ref, o_ref):
    a = a_ref[...].astype(jnp.bfloat16)
    b = b_ref[...].astype(jnp.bfloat16)
    o_ref[...] = lax.dot_general(
        a, b,
        dimension_numbers=(((1,), (1,)), ((), ())),
        preferred_element_type=jnp.float32,
    )


def _tc_matmul(a, b):
    return pl.pallas_call(
        _mm_body,
        grid=(BATCH // _TM,),
        in_specs=[
            pl.BlockSpec((_TM, EMBED), lambda i: (i, 0)),
            pl.BlockSpec((BATCH, EMBED), lambda i: (0, 0)),
        ],
        out_specs=pl.BlockSpec((_TM, BATCH), lambda i: (i, 0)),
        out_shape=jax.ShapeDtypeStruct((BATCH, BATCH), jnp.float32),
    )(a, b)


def kernel(center_word, context_word, W_center, W_context):
    ce, cx = _sc_gather(W_center.T, center_word.astype(jnp.int32),
                        W_context.T, context_word.astype(jnp.int32))
    return _tc_matmul(ce, cx)


# speedup vs baseline: 1.0158x; 1.0158x over previous
import functools

import jax
import jax.numpy as jnp
from jax import lax
from jax.experimental import pallas as pl
from jax.experimental.pallas import tpu as pltpu
from jax.experimental.pallas import tpu_sc as plsc

VOCAB = 1000000
EMBED = 64
BATCH = 4096

_NC = 2
_NS = 16
_NW = _NC * _NS
_BPW = BATCH // _NW
_L = 16
_CH = 4
_NCH = 8


def _gather_one_table(wt_hbm, idx_hbm, out_hbm, base, idx_v, off_v, lane_v,
                      bufs, rows_v, sem):
    pltpu.sync_copy(idx_hbm.at[pl.ds(base, _BPW)], idx_v)
    lanes16 = lax.iota(jnp.int32, _L)

    for i in range(_BPW // _L):
        v = idx_v[pl.ds(i * _L, _L)]
        off_v[pl.ds(i * _L, _L)] = (v >> 7) << 7
        lane_v[pl.ds(i * _L, _L)] = v & 127

    def fire(j, c):
        vo = off_v[pl.ds(j * 32 + (c // 4) * _L, _L)]
        for k in range(_CH):
            off = pl.multiple_of(vo[(c % 4) * _CH + k], 128)
            pltpu.async_copy(wt_hbm.at[:, pl.ds(off, 128)],
                             bufs.at[(c % 2) * _CH + k], sem)

    def run(j, _):
        fire(j, 0)
        for c in range(_NCH):
            if c + 1 < _NCH:
                fire(j, c + 1)
            for k in range(_CH):
                pltpu.make_async_copy(wt_hbm.at[:, pl.ds(0, 128)],
                                      bufs.at[(c % 2) * _CH + k], sem).wait()
            vl = lane_v[pl.ds(j * 32 + (c // 4) * _L, _L)]
            for k in range(_CH):
                i = j * 32 + c * _CH + k
                lane = jnp.full((_L,), vl[(c % 4) * _CH + k], jnp.int32)
                slot = jnp.full((_L,), (c % 2) * _CH + k, jnp.int32)
                for q in range(EMBED // _L):
                    vals = plsc.load_gather(
                        bufs, [slot, lanes16 + q * _L, lane])
                    rows_v[i, pl.ds(q * _L, _L)] = vals
        return ()

    lax.fori_loop(0, _BPW // 32, run, ())
    pltpu.sync_copy(rows_v, out_hbm.at[pl.ds(base, _BPW)])


@functools.partial(
    pl.kernel,
    out_type=(
        jax.ShapeDtypeStruct((BATCH, EMBED), jnp.float32),
        jax.ShapeDtypeStruct((BATCH, EMBED), jnp.float32),
    ),
    mesh=plsc.VectorSubcoreMesh(core_axis_name="c", subcore_axis_name="s"),
    compiler_params=pltpu.CompilerParams(needs_layout_passes=False),
    scratch_types=[
        pltpu.VMEM((_BPW,), jnp.int32),
        pltpu.VMEM((_BPW,), jnp.int32),
        pltpu.VMEM((_BPW,), jnp.int32),
        pltpu.VMEM((2 * _CH, EMBED, 128), jnp.float32),
        pltpu.VMEM((_BPW, EMBED), jnp.float32),
        pltpu.SemaphoreType.DMA,
    ],
)
def _sc_gather(wct_hbm, ci_hbm, wxt_hbm, xi_hbm, out_c, out_x,
               idx_v, off_v, lane_v, bufs, rows_v, sem):
    wid = lax.axis_index("s") * _NC + lax.axis_index("c")
    base = wid * _BPW
    _gather_one_table(wct_hbm, ci_hbm, out_c, base, idx_v, off_v, lane_v,
                      bufs, rows_v, sem)
    _gather_one_table(wxt_hbm, xi_hbm, out_x, base, idx_v, off_v, lane_v,
                      bufs, rows_v, sem)


_TM = 512


def _mm_body(a_ref, b_ref, o_ref):
    a = a_ref[...].astype(jnp.bfloat16)
    b = b_ref[...].astype(jnp.bfloat16)
    o_ref[...] = lax.dot_general(
        a, b,
        dimension_numbers=(((1,), (1,)), ((), ())),
        preferred_element_type=jnp.float32,
    )


def _tc_matmul(a, b):
    return pl.pallas_call(
        _mm_body,
        grid=(BATCH // _TM,),
        in_specs=[
            pl.BlockSpec((_TM, EMBED), lambda i: (i, 0)),
            pl.BlockSpec((BATCH, EMBED), lambda i: (0, 0)),
        ],
        out_specs=pl.BlockSpec((_TM, BATCH), lambda i: (i, 0)),
        out_shape=jax.ShapeDtypeStruct((BATCH, BATCH), jnp.float32),
    )(a, b)


def kernel(center_word, context_word, W_center, W_context):
    ce, cx = _sc_gather(W_center.T, center_word.astype(jnp.int32),
                        W_context.T, context_word.astype(jnp.int32))
    return _tc_matmul(ce, cx)
